# baseline (speedup 1.0000x reference)
import functools

import jax
import jax.numpy as jnp
from jax import lax
from jax.experimental import pallas as pl
from jax.experimental.pallas import tpu as pltpu
from jax.experimental.pallas import tpu_sc as plsc

R = 2048
H = 2048
B = 16
VOCAB = 1000
NSUB = 16
ROWS = R // NSUB
W = ROWS // 32
HW = W // 2
CB = 16
NB = H // CB
STEPS = NB + NSUB - 1


def _column_step(vp, vn, eq, hinp, hinn):
    nw = len(vp)
    x = [eq[w] | vn[w] for w in range(nw)]
    x[0] = x[0] | hinn
    d0 = [None] * nw
    carry = None
    for w in range(nw):
        a = vp[w]
        bb = x[w] & a
        t = a + bb
        s = t if carry is None else t + carry
        if w < nw - 1:
            ov = t.astype(jnp.uint32) < a.astype(jnp.uint32)
            if carry is not None:
                ov = ov | (s.astype(jnp.uint32) < t.astype(jnp.uint32))
            carry = ov.astype(jnp.int32)
        d0[w] = (s ^ a) | x[w]
    hn = [vp[w] & d0[w] for w in range(nw)]
    hp = [vn[w] | ~(vp[w] | d0[w]) for w in range(nw)]
    houtp = lax.shift_right_logical(hp[nw - 1], 31)
    houtn = lax.shift_right_logical(hn[nw - 1], 31)
    vp2 = [None] * nw
    vn2 = [None] * nw
    upp, upn = hinp, hinn
    for w in range(nw):
        shp = (hp[w] << 1) | upp
        shn = (hn[w] << 1) | upn
        if w < nw - 1:
            upp = lax.shift_right_logical(hp[w], 31)
            upn = lax.shift_right_logical(hn[w], 31)
        vp2[w] = shn | ~(shp | d0[w])
        vn2[w] = shp & d0[w]
    return vp2, vn2, houtp, houtn


def _body(ref_hbm, hyp_hbm, out_hbm, ref_v, hyp_v, peq_v, vpn_v, score_v,
          bnd_in, bnd_out, out_v, spmem):
    cid = lax.axis_index("c")
    sid = lax.axis_index("s")
    iota = lax.iota(jnp.int32, 16)
    zero = jnp.zeros((B,), jnp.int32)
    ones = zero - 1

    pltpu.sync_copy(ref_hbm.at[pl.ds(sid * (ROWS * B), ROWS * B)], ref_v)
    pltpu.sync_copy(hyp_hbm, hyp_v)

    def zero_peq(i, _):
        peq_v[pl.ds(i * B, B)] = zero
        return 0

    lax.fori_loop(0, VOCAB * W, zero_peq, 0, unroll=8)

    for w in range(W):
        def set_bit(r2, _, w=w):
            sym = ref_v[pl.ds((w * 32 + r2) * B, B)]
            idx = (sym * (W * B)) + (w * B) + iota
            bit = (zero + 1) << r2
            cur = plsc.load_gather(peq_v, [idx])
            plsc.store_scatter(peq_v, [idx], cur | bit)
            return 0

        lax.fori_loop(0, 32, set_bit, 0)

    for w in range(W):
        vpn_v[pl.ds(w * B, B)] = ones
        vpn_v[pl.ds((W + w) * B, B)] = zero
    score_v[...] = zero + R

    def step(k, _):
        b = k - sid
        valid = jnp.logical_and(b >= 0, b < NB)

        @pl.when(jnp.logical_and(valid, sid > 0))
        def _():
            slot = ((k + 1) % 2) * NSUB + (sid - 1)
            pltpu.sync_copy(spmem.at[pl.ds(slot * (2 * B), 2 * B)], bnd_in)

        @pl.when(jnp.logical_and(valid, sid == 0))
        def _():
            bnd_in[pl.ds(0, B)] = ones
            bnd_in[pl.ds(B, B)] = zero

        @pl.when(valid)
        def _():
            hinp_pack = bnd_in[pl.ds(0, B)]
            hinn_pack = bnd_in[pl.ds(B, B)]
            vp = [vpn_v[pl.ds(w * B, B)] for w in range(W)]
            vn = [vpn_v[pl.ds((W + w) * B, B)] for w in range(W)]

            def upper_col(jj):
                hv = hyp_v[pl.ds((b * CB + jj) * B, B)]
                idx0 = (hv * (W * B)) + iota
                hinp = lax.shift_right_logical(hinp_pack, jj) & 1
                hinn = lax.shift_right_logical(hinn_pack, jj) & 1
                return idx0, hinp, hinn

            idx0, hinp, hinn = upper_col(0)
            equ = [plsc.load_gather(peq_v, [idx0 + (w * B)])
                   for w in range(HW)]
            vpu, vnu, hopu, honu = _column_step(
                vp[:HW], vn[:HW], equ, hinp, hinn)

            def slot(t, carry):
                vpu, vnu, vpl, vnl, score, outp, outn, hopu, honu, idxp = \
                    carry
                idx0, hinp, hinn = upper_col(t)
                equ = [plsc.load_gather(peq_v, [idx0 + (w * B)])
                       for w in range(HW)]
                nvpu, nvnu, nhopu, nhonu = _column_step(
                    list(vpu), list(vnu), equ, hinp, hinn)
                eql = [plsc.load_gather(peq_v, [idxp + ((HW + w) * B)])
                       for w in range(W - HW)]
                nvpl, nvnl, hop, hon = _column_step(
                    list(vpl), list(vnl), eql, hopu, honu)
                score = score + hop - hon
                outp = outp | (hop << (t - 1))
                outn = outn | (hon << (t - 1))
                return (tuple(nvpu), tuple(nvnu), tuple(nvpl), tuple(nvnl),
                        score, outp, outn, nhopu, nhonu, idx0)

            vpu, vnu, vpl, vnl, score, outp, outn, hopu, honu, idxp = \
                lax.fori_loop(
                    1, CB, slot,
                    (tuple(vpu), tuple(vnu),
                     tuple(vp[HW:]), tuple(vn[HW:]),
                     score_v[...], zero, zero, hopu, honu, idx0))

            eql = [plsc.load_gather(peq_v, [idxp + ((HW + w) * B)])
                   for w in range(W - HW)]
            vpl, vnl, hop, hon = _column_step(
                list(vpl), list(vnl), eql, hopu, honu)
            score = score + hop - hon
            outp = outp | (hop << (CB - 1))
            outn = outn | (hon << (CB - 1))

            vp = list(vpu) + list(vpl)
            vn = list(vnu) + list(vnl)
            for w in range(W):
                vpn_v[pl.ds(w * B, B)] = vp[w]
                vpn_v[pl.ds((W + w) * B, B)] = vn[w]
            score_v[...] = score
            bnd_out[pl.ds(0, B)] = outp
            bnd_out[pl.ds(B, B)] = outn
            slot = (k % 2) * NSUB + sid
            pltpu.sync_copy(bnd_out, spmem.at[pl.ds(slot * (2 * B), 2 * B)])

        plsc.subcore_barrier()
        return 0

    lax.fori_loop(0, STEPS, step, 0)

    @pl.when(jnp.logical_and(cid == 0, sid == NSUB - 1))
    def _():
        out_v[...] = score_v[...].astype(jnp.float32)
        pltpu.sync_copy(out_v, out_hbm)


@jax.jit
def kernel(ref, hyp):
    mesh = plsc.VectorSubcoreMesh(core_axis_name="c", subcore_axis_name="s")
    f = functools.partial(
        pl.kernel,
        mesh=mesh,
        compiler_params=pltpu.CompilerParams(needs_layout_passes=False),
        out_type=jax.ShapeDtypeStruct((B,), jnp.float32),
        scratch_types=[
            pltpu.VMEM((ROWS * B,), jnp.int32),
            pltpu.VMEM((H * B,), jnp.int32),
            pltpu.VMEM((VOCAB * W * B,), jnp.int32),
            pltpu.VMEM((2 * W * B,), jnp.int32),
            pltpu.VMEM((B,), jnp.int32),
            pltpu.VMEM((2 * B,), jnp.int32),
            pltpu.VMEM((2 * B,), jnp.int32),
            pltpu.VMEM((B,), jnp.float32),
            pltpu.VMEM_SHARED((2 * NSUB * 2 * B,), jnp.int32),
        ],
    )(_body)
    return f(ref.reshape(R * B), hyp.reshape(H * B))


# speedup vs baseline: 1.0824x; 1.0824x over previous
import functools

import jax
import jax.numpy as jnp
from jax import lax
from jax.experimental import pallas as pl
from jax.experimental.pallas import tpu as pltpu
from jax.experimental.pallas import tpu_sc as plsc

R = 2048
H = 2048
B = 16
VOCAB = 1000
NSUB = 16
ROWS = R // NSUB
W = ROWS // 32
HW = W // 2
CB = 32
NB = H // CB
STEPS = NB + NSUB - 1


def _column_step(vp, vn, eq, hinp, hinn):
    nw = len(vp)
    x = [eq[w] | vn[w] for w in range(nw)]
    x[0] = x[0] | hinn
    d0 = [None] * nw
    carry = None
    for w in range(nw):
        a = vp[w]
        bb = x[w] & a
        t = a + bb
        s = t if carry is None else t + carry
        if w < nw - 1:
            ov = t.astype(jnp.uint32) < a.astype(jnp.uint32)
            if carry is not None:
                ov = ov | (s.astype(jnp.uint32) < t.astype(jnp.uint32))
            carry = ov.astype(jnp.int32)
        d0[w] = (s ^ a) | x[w]
    hn = [vp[w] & d0[w] for w in range(nw)]
    hp = [vn[w] | ~(vp[w] | d0[w]) for w in range(nw)]
    houtp = lax.shift_right_logical(hp[nw - 1], 31)
    houtn = lax.shift_right_logical(hn[nw - 1], 31)
    vp2 = [None] * nw
    vn2 = [None] * nw
    upp, upn = hinp, hinn
    for w in range(nw):
        shp = (hp[w] << 1) | upp
        shn = (hn[w] << 1) | upn
        if w < nw - 1:
            upp = lax.shift_right_logical(hp[w], 31)
            upn = lax.shift_right_logical(hn[w], 31)
        vp2[w] = shn | ~(shp | d0[w])
        vn2[w] = shp & d0[w]
    return vp2, vn2, houtp, houtn


def _body(ref_hbm, hyp_hbm, out_hbm, ref_v, hyp_v, peq_v, vpn_v, score_v,
          bnd_in, bnd_out, out_v, spmem):
    cid = lax.axis_index("c")
    sid = lax.axis_index("s")
    iota = lax.iota(jnp.int32, 16)
    zero = jnp.zeros((B,), jnp.int32)
    ones = zero - 1

    pltpu.sync_copy(ref_hbm.at[pl.ds(sid * (ROWS * B), ROWS * B)], ref_v)
    pltpu.sync_copy(hyp_hbm, hyp_v)

    def zero_peq(i, _):
        peq_v[pl.ds(i * B, B)] = zero
        return 0

    lax.fori_loop(0, VOCAB * W, zero_peq, 0, unroll=8)

    for w in range(W):
        def set_bit(r2, _, w=w):
            sym = ref_v[pl.ds((w * 32 + r2) * B, B)]
            idx = (sym * (W * B)) + (w * B) + iota
            bit = (zero + 1) << r2
            cur = plsc.load_gather(peq_v, [idx])
            plsc.store_scatter(peq_v, [idx], cur | bit)
            return 0

        lax.fori_loop(0, 32, set_bit, 0)

    for w in range(W):
        vpn_v[pl.ds(w * B, B)] = ones
        vpn_v[pl.ds((W + w) * B, B)] = zero
    score_v[...] = zero + R

    def step(k, _):
        b = k - sid
        valid = jnp.logical_and(b >= 0, b < NB)

        @pl.when(jnp.logical_and(valid, sid > 0))
        def _():
            slot = ((k + 1) % 2) * NSUB + (sid - 1)
            pltpu.sync_copy(spmem.at[pl.ds(slot * (2 * B), 2 * B)], bnd_in)

        @pl.when(jnp.logical_and(valid, sid == 0))
        def _():
            bnd_in[pl.ds(0, B)] = ones
            bnd_in[pl.ds(B, B)] = zero

        @pl.when(valid)
        def _():
            hinp_pack = bnd_in[pl.ds(0, B)]
            hinn_pack = bnd_in[pl.ds(B, B)]
            vp = [vpn_v[pl.ds(w * B, B)] for w in range(W)]
            vn = [vpn_v[pl.ds((W + w) * B, B)] for w in range(W)]

            def upper_col(jj):
                hv = hyp_v[pl.ds((b * CB + jj) * B, B)]
                idx0 = (hv * (W * B)) + iota
                hinp = lax.shift_right_logical(hinp_pack, jj) & 1
                hinn = lax.shift_right_logical(hinn_pack, jj) & 1
                return idx0, hinp, hinn

            idx0, hinp, hinn = upper_col(0)
            equ = [plsc.load_gather(peq_v, [idx0 + (w * B)])
                   for w in range(HW)]
            vpu, vnu, hopu, honu = _column_step(
                vp[:HW], vn[:HW], equ, hinp, hinn)

            def slot(t, carry):
                vpu, vnu, vpl, vnl, score, outp, outn, hopu, honu, idxp = \
                    carry
                idx0, hinp, hinn = upper_col(t)
                equ = [plsc.load_gather(peq_v, [idx0 + (w * B)])
                       for w in range(HW)]
                nvpu, nvnu, nhopu, nhonu = _column_step(
                    list(vpu), list(vnu), equ, hinp, hinn)
                eql = [plsc.load_gather(peq_v, [idxp + ((HW + w) * B)])
                       for w in range(W - HW)]
                nvpl, nvnl, hop, hon = _column_step(
                    list(vpl), list(vnl), eql, hopu, honu)
                score = score + hop - hon
                outp = outp | (hop << (t - 1))
                outn = outn | (hon << (t - 1))
                return (tuple(nvpu), tuple(nvnu), tuple(nvpl), tuple(nvnl),
                        score, outp, outn, nhopu, nhonu, idx0)

            vpu, vnu, vpl, vnl, score, outp, outn, hopu, honu, idxp = \
                lax.fori_loop(
                    1, CB, slot,
                    (tuple(vpu), tuple(vnu),
                     tuple(vp[HW:]), tuple(vn[HW:]),
                     score_v[...], zero, zero, hopu, honu, idx0))

            eql = [plsc.load_gather(peq_v, [idxp + ((HW + w) * B)])
                   for w in range(W - HW)]
            vpl, vnl, hop, hon = _column_step(
                list(vpl), list(vnl), eql, hopu, honu)
            score = score + hop - hon
            outp = outp | (hop << (CB - 1))
            outn = outn | (hon << (CB - 1))

            vp = list(vpu) + list(vpl)
            vn = list(vnu) + list(vnl)
            for w in range(W):
                vpn_v[pl.ds(w * B, B)] = vp[w]
                vpn_v[pl.ds((W + w) * B, B)] = vn[w]
            score_v[...] = score
            bnd_out[pl.ds(0, B)] = outp
            bnd_out[pl.ds(B, B)] = outn
            slot = (k % 2) * NSUB + sid
            pltpu.sync_copy(bnd_out, spmem.at[pl.ds(slot * (2 * B), 2 * B)])

        plsc.subcore_barrier()
        return 0

    lax.fori_loop(0, STEPS, step, 0)

    @pl.when(jnp.logical_and(cid == 0, sid == NSUB - 1))
    def _():
        out_v[...] = score_v[...].astype(jnp.float32)
        pltpu.sync_copy(out_v, out_hbm)


@jax.jit
def kernel(ref, hyp):
    mesh = plsc.VectorSubcoreMesh(core_axis_name="c", subcore_axis_name="s")
    f = functools.partial(
        pl.kernel,
        mesh=mesh,
        compiler_params=pltpu.CompilerParams(needs_layout_passes=False),
        out_type=jax.ShapeDtypeStruct((B,), jnp.float32),
        scratch_types=[
            pltpu.VMEM((ROWS * B,), jnp.int32),
            pltpu.VMEM((H * B,), jnp.int32),
            pltpu.VMEM((VOCAB * W * B,), jnp.int32),
            pltpu.VMEM((2 * W * B,), jnp.int32),
            pltpu.VMEM((B,), jnp.int32),
            pltpu.VMEM((2 * B,), jnp.int32),
            pltpu.VMEM((2 * B,), jnp.int32),
            pltpu.VMEM((B,), jnp.float32),
            pltpu.VMEM_SHARED((2 * NSUB * 2 * B,), jnp.int32),
        ],
    )(_body)
    return f(ref.reshape(R * B), hyp.reshape(H * B))
